# baseline (speedup 1.0000x reference)
import functools

import jax
import jax.numpy as jnp
import numpy as np
from jax import lax
from jax.experimental import pallas as pl
from jax.experimental.pallas import tpu as pltpu
from jax.experimental.pallas import tpu_sc as plsc

N = 10000
E = 320000
D = 128
H = 8
C = 16
ED = 10
FF = 4 * D

_BN = 1000


def _proj_body(x_ref, wq, bq, wk, bk, wv, bv, ws, bs, q_o, kv_o, xr_o):
    xb = x_ref[...]
    q_o[...] = jnp.dot(xb, wq[...], preferred_element_type=jnp.float32) + bq[...]
    kv_o[:, :D] = jnp.dot(xb, wk[...], preferred_element_type=jnp.float32) + bk[...]
    kv_o[:, D:] = jnp.dot(xb, wv[...], preferred_element_type=jnp.float32) + bv[...]
    xr_o[...] = jnp.dot(xb, ws[...], preferred_element_type=jnp.float32) + bs[...]


def _proj(x, wq, bq, wk, bk, wv, bv, ws, bs):
    blk = pl.BlockSpec((_BN, D), lambda i: (i, 0))
    wspec = pl.BlockSpec((D, D), lambda i: (0, 0))
    bspec = pl.BlockSpec((1, D), lambda i: (0, 0))
    return pl.pallas_call(
        _proj_body,
        grid=(N // _BN,),
        in_specs=[blk, wspec, bspec, wspec, bspec, wspec, bspec, wspec, bspec],
        out_specs=[blk, pl.BlockSpec((_BN, 2 * D), lambda i: (i, 0)), blk],
        out_shape=[jax.ShapeDtypeStruct((N, D), jnp.float32),
                   jax.ShapeDtypeStruct((N, 2 * D), jnp.float32),
                   jax.ShapeDtypeStruct((N, D), jnp.float32)],
    )(x, wq, bq, wk, bk, wv, bv, ws, bs)


_BE = 2560


def _eproj_body(eat_ref, we, be, o_ref):
    o_ref[...] = lax.dot_general(
        eat_ref[...], we[...], (((0,), (0,)), ((), ())),
        preferred_element_type=jnp.float32) + be[...]


def _eproj(eat, we, be):
    return pl.pallas_call(
        _eproj_body,
        grid=(E // _BE,),
        in_specs=[
            pl.BlockSpec((ED, _BE), lambda i: (0, i)),
            pl.BlockSpec((ED, D), lambda i: (0, 0)),
            pl.BlockSpec((1, D), lambda i: (0, 0)),
        ],
        out_specs=pl.BlockSpec((_BE, D), lambda i: (i, 0)),
        out_shape=jax.ShapeDtypeStruct((E, D), jnp.float32),
    )(eat, we, be)


_GDN = lax.GatherDimensionNumbers(
    offset_dims=(), collapsed_slice_dims=(0,), start_index_map=(0,))


def _shuf(v, idx):
    return lax.gather(v, idx[:, None], _GDN, slice_sizes=(1,),
                      mode=lax.GatherScatterMode.PROMISE_IN_BOUNDS)


_NTILE = 16
_NW = 2 * _NTILE
_EPW = E // _NW
_CH = 16
_NCH = _EPW // _CH
_NP = 10240
_RPT = _NP // _NTILE
_DR = _NP // 8
_DRT = _DR // _NTILE

_LANE_OF = (0, 8, 4, 12, 2, 10, 6, 14)


def _edge_kernel_body(q_hbm, kv_hbm, ep_hbm, src_hbm, dst_hbm, z_hbm,
                      num_hbm, den_hbm,
                      src0, src1, src2, dst0, dst1, dst2,
                      dsts0, dsts1, dsts2, dhi0, dhi1, dhi2,
                      cv0, cv1, cv2,
                      qb0, qb1, qb2, kvb0, kvb1, kvb2, eb0, eb1, eb2,
                      mb0, mb1, mb2, db0, db1, db2, nacc, dacc,
                      isem0, isem1, isem2, gsem0, gsem1, gsem2,
                      ssem0, ssem1, ssem2):
    cid = lax.axis_index("c")
    sid = lax.axis_index("s")
    wid = cid * _NTILE + sid
    ebase = wid * _EPW

    srcv = (src0, src1, src2)
    dstv = (dst0, dst1, dst2)
    dstsv = (dsts0, dsts1, dsts2)
    dhiv = (dhi0, dhi1, dhi2)
    colvb = (cv0, cv1, cv2)
    qb_ = (qb0, qb1, qb2)
    kvb_ = (kvb0, kvb1, kvb2)
    eb_ = (eb0, eb1, eb2)
    mb_ = (mb0, mb1, mb2)
    db_ = (db0, db1, db2)
    isem = (isem0, isem1, isem2)
    gsem = (gsem0, gsem1, gsem2)
    ssem = (ssem0, ssem1, ssem2)

    pltpu.sync_copy(z_hbm.at[pl.ds(0, _RPT)],
                    nacc.at[pl.ds(sid * _RPT, _RPT)])
    pltpu.sync_copy(z_hbm.at[pl.ds(0, _DRT)], dacc.at[pl.ds(sid * _DRT, _DRT)])
    zero16 = jnp.zeros((C,), jnp.float32)
    zi16 = jnp.zeros((C,), jnp.int32)
    for j in range(_CH):
        for bq in range(H):
            db0[j, pl.ds(bq * C, C)] = zero16
            db1[j, pl.ds(bq * C, C)] = zero16
            db2[j, pl.ds(bq * C, C)] = zero16
    cv0[...] = zi16
    cv1[...] = zi16
    cv2[...] = zi16
    plsc.subcore_barrier()

    lane = lax.iota(jnp.int32, C)
    rp = {sh: (lane + sh) % C for sh in (1, 2, 4, 8)}
    rm = {sh: (lane - sh) % C for sh in (2, 4)}
    msk = {sh: (lane & sh) == 0 for sh in (2, 4, 8)}

    def comb(A, B, sh):
        rB = rp[sh] if sh == 8 else rm[sh]
        return jnp.where(msk[sh], A + _shuf(A, rp[sh]), B + _shuf(B, rB))

    def issue_idx(sl, cc):
        base = ebase + cc * _CH
        pltpu.async_copy(src_hbm.at[pl.ds(base, _CH)], srcv[sl], isem[sl])
        pltpu.async_copy(dst_hbm.at[pl.ds(base, _CH)], dstv[sl], isem[sl])

    def wait_idx(sl):
        pltpu.make_async_copy(src_hbm.at[pl.ds(0, _CH)], srcv[sl],
                              isem[sl]).wait()
        pltpu.make_async_copy(dst_hbm.at[pl.ds(0, _CH)], dstv[sl],
                              isem[sl]).wait()

    def issue_gather(sl, cc):
        base = ebase + cc * _CH
        pltpu.async_copy(kv_hbm.at[srcv[sl]], kvb_[sl], gsem[sl])
        pltpu.async_copy(q_hbm.at[dstv[sl]], qb_[sl], gsem[sl])
        pltpu.async_copy(ep_hbm.at[pl.ds(base, _CH)], eb_[sl], gsem[sl])

    def wait_gather(sl):
        pltpu.make_async_copy(kv_hbm.at[srcv[sl]], kvb_[sl], gsem[sl]).wait()
        pltpu.make_async_copy(q_hbm.at[dstv[sl]], qb_[sl], gsem[sl]).wait()
        pltpu.make_async_copy(ep_hbm.at[pl.ds(0, _CH)], eb_[sl],
                              gsem[sl]).wait()

    def issue_scatter(sl):
        pltpu.async_copy(mb_[sl], nacc.at[dstsv[sl]], ssem[sl], add=True)
        pltpu.async_copy(db_[sl], dacc.at[dhiv[sl]], ssem[sl], add=True)

    def wait_scatter(sl):
        pltpu.make_async_copy(mb_[sl], nacc.at[dstsv[sl]], ssem[sl]).wait()
        pltpu.make_async_copy(db_[sl], dacc.at[dhiv[sl]], ssem[sl]).wait()

    def compute(sl, colv):
        qb, kvb, eb, mb, db = (qb_[sl], kvb_[sl], eb_[sl], mb_[sl], db_[sl])
        oldcolv = colvb[sl][...]
        colvb[sl][...] = colv
        for j in range(_CH):
            ps = []
            us = []
            for h in range(H):
                slc = pl.ds(h * C, C)
                e = eb[j, slc]
                ps.append(qb[j, slc] * (kvb[j, slc] + e))
                us.append(kvb[j, pl.ds(D + h * C, C)] + e)
            z1 = comb(ps[0], ps[1], 8)
            z2 = comb(ps[2], ps[3], 8)
            z3 = comb(ps[4], ps[5], 8)
            z4 = comb(ps[6], ps[7], 8)
            za = comb(z1, z2, 4)
            zb = comb(z3, z4, 4)
            zf = comb(za, zb, 2)
            av = zf + _shuf(zf, rp[1])
            ex = jnp.exp(av)
            db[j, pl.ds(oldcolv[j], C)] = zero16
            db[j, pl.ds(colv[j], C)] = ex
            for h in range(H):
                mb[j, pl.ds(h * C, C)] = us[h] * ex[_LANE_OF[h]]

    def chunk_step(cc, sl, g2ok, i3ok, wsok):
        if g2ok:
            wait_idx((sl + 2) % 3)
            issue_gather((sl + 2) % 3, cc + 2)
        if wsok:
            wait_scatter(sl)
        wait_gather(sl)
        d16 = dstv[sl][...]
        dstsv[sl][...] = d16
        dhiv[sl][...] = lax.shift_right_logical(d16, 3)
        colv = (d16 & 7) * C
        if i3ok:
            issue_idx(sl, cc + 3)
        compute(sl, colv)
        issue_scatter(sl)

    def guarded_step(cc, sl):
        @pl.when(cc + 2 <= _NCH - 1)
        def _():
            wait_idx((sl + 2) % 3)
            issue_gather((sl + 2) % 3, cc + 2)

        @pl.when(cc >= 3)
        def _():
            wait_scatter(sl)
        wait_gather(sl)
        d16 = dstv[sl][...]
        dstsv[sl][...] = d16
        dhiv[sl][...] = lax.shift_right_logical(d16, 3)
        colv = (d16 & 7) * C

        @pl.when(cc + 3 <= _NCH - 1)
        def _():
            issue_idx(sl, cc + 3)
        compute(sl, colv)
        issue_scatter(sl)

    issue_idx(0, 0)
    issue_idx(1, 1)
    issue_idx(2, 2)
    wait_idx(0)
    issue_gather(0, 0)
    wait_idx(1)
    issue_gather(1, 1)
    chunk_step(0, 0, g2ok=True, i3ok=True, wsok=False)

    def triple_body(tt, _):
        cc1 = 1 + 3 * tt
        guarded_step(cc1, 1)
        guarded_step(cc1 + 1, 2)
        guarded_step(cc1 + 2, 0)
        return 0

    lax.fori_loop(0, (_NCH - 1) // 3, triple_body, 0)
    wait_scatter(1)
    wait_scatter(2)
    wait_scatter(0)
    plsc.subcore_barrier()
    pltpu.sync_copy(nacc.at[pl.ds(sid * _RPT, _RPT)],
                    num_hbm.at[cid, pl.ds(sid * _RPT, _RPT)])
    pltpu.sync_copy(dacc.at[pl.ds(sid * _DRT, _DRT)],
                    den_hbm.at[cid, pl.ds(sid * _DRT, _DRT)])


def _edge_attention(q, kv, ep, src, dst):
    mesh = plsc.VectorSubcoreMesh(core_axis_name="c", subcore_axis_name="s")
    zeros_hbm = jnp.zeros((_RPT, D), jnp.float32)
    i16 = lambda: pltpu.VMEM((_CH,), jnp.int32)
    f16 = lambda: pltpu.VMEM((_CH, D), jnp.float32)
    f32w = lambda: pltpu.VMEM((_CH, 2 * D), jnp.float32)
    kern = functools.partial(
        pl.kernel,
        mesh=mesh,
        out_type=[
            jax.ShapeDtypeStruct((2, _NP, D), jnp.float32),
            jax.ShapeDtypeStruct((2, _DR, D), jnp.float32),
        ],
        scratch_types=(
            [i16() for _ in range(15)]
            + [f16() for _ in range(3)] + [f32w() for _ in range(3)]
            + [f16() for _ in range(9)] + [
                pltpu.VMEM_SHARED((_NP, D), jnp.float32),
                pltpu.VMEM_SHARED((_DR, D), jnp.float32),
            ] + [pltpu.SemaphoreType.DMA] * 9
        ),
    )(_edge_kernel_body)
    return kern(q, kv, ep, src, dst, zeros_hbm)


def _post_body(num_ref, den_ref, x_ref, xr_ref, wa, wb, erep, h_o, stats_o):
    i = pl.program_id(0)
    o = num_ref[0] + num_ref[1]
    den = den_ref[0] + den_ref[1]
    den128 = jnp.zeros((_BN, D), jnp.float32)
    for hh in range(H):
        lh = _LANE_OF[hh]
        den128 = den128 + (1.0 / (den[:, lh:lh + 1] + 1e-16)) * erep[hh:hh + 1, :]
    out = o * den128
    xr = xr_ref[...]
    xb = x_ref[...]
    ba = jnp.sum(out * wa[...], axis=1, keepdims=True)
    bb = jnp.sum(xr * wb[...], axis=1, keepdims=True)
    b = jax.nn.sigmoid(ba + bb)
    hcur = xb + b * xr + (1.0 - b) * out
    h_o[...] = hcur

    @pl.when(i == 0)
    def _():
        stats_o[...] = jnp.zeros_like(stats_o)
    stats_o[0:1, :] += jnp.sum(hcur, axis=0, keepdims=True)
    stats_o[1:2, :] += jnp.sum(hcur * hcur, axis=0, keepdims=True)


def _post(num, den, x, xr, wa, wb, erep):
    return pl.pallas_call(
        _post_body,
        grid=(N // _BN,),
        in_specs=[
            pl.BlockSpec((2, _BN, D), lambda i: (0, i, 0)),
            pl.BlockSpec((2, _BN, C), lambda i: (0, i, 0)),
            pl.BlockSpec((_BN, D), lambda i: (i, 0)),
            pl.BlockSpec((_BN, D), lambda i: (i, 0)),
            pl.BlockSpec((1, D), lambda i: (0, 0)),
            pl.BlockSpec((1, D), lambda i: (0, 0)),
            pl.BlockSpec((H, D), lambda i: (0, 0)),
        ],
        out_specs=[
            pl.BlockSpec((_BN, D), lambda i: (i, 0)),
            pl.BlockSpec((8, D), lambda i: (0, 0)),
        ],
        out_shape=[
            jax.ShapeDtypeStruct((N, D), jnp.float32),
            jax.ShapeDtypeStruct((8, D), jnp.float32),
        ],
    )(num, den, x, xr, wa, wb, erep)


_SQRT_HALF = 0.7071067811865476


def _ffn_body(h_ref, stats_ref, gnw, gnb, gms, w1, b1, w2, b2, o_ref):
    s1 = stats_ref[0:1, :]
    s2 = stats_ref[1:2, :]
    m = s1 * (1.0 / N)
    q2 = s2 * (1.0 / N)
    g = gms[...]
    var = q2 - (2.0 * g - g * g) * m * m
    inv = lax.rsqrt(var + 1e-5)
    hb = h_ref[...]
    o = hb - g * m
    hn = gnw[...] * o * inv + gnb[...]
    t = jnp.dot(hn, w1[...], preferred_element_type=jnp.float32) + b1[...]
    gel = t * 0.5 * (1.0 + lax.erf(t * _SQRT_HALF))
    f = jnp.dot(gel, w2[...], preferred_element_type=jnp.float32) + b2[...]
    o_ref[...] = hn + f


def _ffn(h, stats, gnw, gnb, gms, w1, b1, w2, b2):
    return pl.pallas_call(
        _ffn_body,
        grid=(N // _BN,),
        in_specs=[
            pl.BlockSpec((_BN, D), lambda i: (i, 0)),
            pl.BlockSpec((8, D), lambda i: (0, 0)),
            pl.BlockSpec((1, D), lambda i: (0, 0)),
            pl.BlockSpec((1, D), lambda i: (0, 0)),
            pl.BlockSpec((1, D), lambda i: (0, 0)),
            pl.BlockSpec((D, FF), lambda i: (0, 0)),
            pl.BlockSpec((1, FF), lambda i: (0, 0)),
            pl.BlockSpec((FF, D), lambda i: (0, 0)),
            pl.BlockSpec((1, D), lambda i: (0, 0)),
        ],
        out_specs=pl.BlockSpec((_BN, D), lambda i: (i, 0)),
        out_shape=jax.ShapeDtypeStruct((N, D), jnp.float32),
    )(h, stats, gnw, gnb, gms, w1, b1, w2, b2)


def kernel(x, edge_index, edge_attr, params):
    p = params
    row = lambda a: a.reshape(1, -1)
    q, kv, xr = _proj(
        x, p['Wq'] * 0.25, row(p['bq'] * 0.25), p['Wk'], row(p['bk']),
        p['Wv'], row(p['bv']), p['Wskip'], row(p['bskip']))
    ep = _eproj(edge_attr.T, p['We'], row(p['be']))
    src = edge_index[0]
    dst = edge_index[1]
    num, den = _edge_attention(q, kv, ep, src, dst)
    den = den.reshape(2, _NP, C)

    wbeta = p['Wbeta'][:, 0]
    wa = row(wbeta[:D] + wbeta[2 * D:])
    wb = row(wbeta[D:2 * D] - wbeta[2 * D:])
    erep = jnp.repeat(jnp.eye(H, dtype=jnp.float32), C, axis=1)
    h, stats = _post(num, den, x, xr, wa, wb, erep)
    return _ffn(h, stats, row(p['gn_weight']), row(p['gn_bias']),
                row(p['gn_mean_scale']), p['W1'], row(p['b1']),
                p['W2'], row(p['b2']))


# speedup vs baseline: 1.0886x; 1.0886x over previous
import functools

import jax
import jax.numpy as jnp
import numpy as np
from jax import lax
from jax.experimental import pallas as pl
from jax.experimental.pallas import tpu as pltpu
from jax.experimental.pallas import tpu_sc as plsc

N = 10000
E = 320000
D = 128
H = 8
C = 16
ED = 10
FF = 4 * D

_BN = 1000


def _proj_body(x_ref, wq, bq, wk, bk, wv, bv, ws, bs, q_o, kv_o, xr_o):
    xb = x_ref[...]
    q_o[...] = jnp.dot(xb, wq[...], preferred_element_type=jnp.float32) + bq[...]
    kv_o[:, :D] = jnp.dot(xb, wk[...], preferred_element_type=jnp.float32) + bk[...]
    kv_o[:, D:] = jnp.dot(xb, wv[...], preferred_element_type=jnp.float32) + bv[...]
    xr_o[...] = jnp.dot(xb, ws[...], preferred_element_type=jnp.float32) + bs[...]


def _proj(x, wq, bq, wk, bk, wv, bv, ws, bs):
    blk = pl.BlockSpec((_BN, D), lambda i: (i, 0))
    wspec = pl.BlockSpec((D, D), lambda i: (0, 0))
    bspec = pl.BlockSpec((1, D), lambda i: (0, 0))
    return pl.pallas_call(
        _proj_body,
        grid=(N // _BN,),
        in_specs=[blk, wspec, bspec, wspec, bspec, wspec, bspec, wspec, bspec],
        out_specs=[blk, pl.BlockSpec((_BN, 2 * D), lambda i: (i, 0)), blk],
        out_shape=[jax.ShapeDtypeStruct((N, D), jnp.float32),
                   jax.ShapeDtypeStruct((N, 2 * D), jnp.float32),
                   jax.ShapeDtypeStruct((N, D), jnp.float32)],
    )(x, wq, bq, wk, bk, wv, bv, ws, bs)


_BE = 2560


def _eproj_body(eat_ref, we, be, o_ref):
    o_ref[...] = lax.dot_general(
        eat_ref[...], we[...], (((0,), (0,)), ((), ())),
        preferred_element_type=jnp.float32) + be[...]


def _eproj(eat, we, be):
    return pl.pallas_call(
        _eproj_body,
        grid=(E // _BE,),
        in_specs=[
            pl.BlockSpec((ED, _BE), lambda i: (0, i)),
            pl.BlockSpec((ED, D), lambda i: (0, 0)),
            pl.BlockSpec((1, D), lambda i: (0, 0)),
        ],
        out_specs=pl.BlockSpec((_BE, D), lambda i: (i, 0)),
        out_shape=jax.ShapeDtypeStruct((E, D), jnp.float32),
    )(eat, we, be)


_GDN = lax.GatherDimensionNumbers(
    offset_dims=(), collapsed_slice_dims=(0,), start_index_map=(0,))


def _shuf(v, idx):
    return lax.gather(v, idx[:, None], _GDN, slice_sizes=(1,),
                      mode=lax.GatherScatterMode.PROMISE_IN_BOUNDS)


_NTILE = 16
_NW = 2 * _NTILE
_EPW = E // _NW
_CH = 16
_NCH = _EPW // _CH
_NP = 10240
_RPT = _NP // _NTILE
_DR = _NP // 8
_DRT = _DR // _NTILE

_LANE_OF = (0, 8, 4, 12, 2, 10, 6, 14)


def _edge_kernel_body(q_hbm, kv_hbm, ep_hbm, src_hbm, dst_hbm, z_hbm,
                      num_hbm, den_hbm,
                      src0, src1, dst0, dst1, dsts0, dsts1, dhi0, dhi1,
                      cv0, cv1,
                      qb0, qb1, kvb0, kvb1, eb0, eb1,
                      mb0, mb1, db0, db1, nacc, dacc,
                      isem0, isem1, gsem0, gsem1, ssem0, ssem1):
    cid = lax.axis_index("c")
    sid = lax.axis_index("s")
    wid = cid * _NTILE + sid
    ebase = wid * _EPW

    srcv = (src0, src1)
    dstv = (dst0, dst1)
    dstsv = (dsts0, dsts1)
    dhiv = (dhi0, dhi1)
    colvb = (cv0, cv1)
    qb_ = (qb0, qb1)
    kvb_ = (kvb0, kvb1)
    eb_ = (eb0, eb1)
    mb_ = (mb0, mb1)
    db_ = (db0, db1)
    isem = (isem0, isem1)
    gsem = (gsem0, gsem1)
    ssem = (ssem0, ssem1)

    pltpu.sync_copy(z_hbm.at[pl.ds(0, _RPT)],
                    nacc.at[pl.ds(sid * _RPT, _RPT)])
    pltpu.sync_copy(z_hbm.at[pl.ds(0, _DRT)], dacc.at[pl.ds(sid * _DRT, _DRT)])
    zero16 = jnp.zeros((C,), jnp.float32)
    zi16 = jnp.zeros((C,), jnp.int32)
    for j in range(_CH):
        for bq in range(H):
            db0[j, pl.ds(bq * C, C)] = zero16
            db1[j, pl.ds(bq * C, C)] = zero16
    cv0[...] = zi16
    cv1[...] = zi16
    plsc.subcore_barrier()

    lane = lax.iota(jnp.int32, C)
    rp = {sh: (lane + sh) % C for sh in (1, 2, 4, 8)}
    rm = {sh: (lane - sh) % C for sh in (2, 4)}
    msk = {sh: (lane & sh) == 0 for sh in (2, 4, 8)}

    def comb(A, B, sh):
        rB = rp[sh] if sh == 8 else rm[sh]
        return jnp.where(msk[sh], A + _shuf(A, rp[sh]), B + _shuf(B, rB))

    def issue_idx(sl, cc):
        base = ebase + cc * _CH
        pltpu.async_copy(src_hbm.at[pl.ds(base, _CH)], srcv[sl], isem[sl])
        pltpu.async_copy(dst_hbm.at[pl.ds(base, _CH)], dstv[sl], isem[sl])

    def wait_idx(sl):
        pltpu.make_async_copy(src_hbm.at[pl.ds(0, _CH)], srcv[sl],
                              isem[sl]).wait()
        pltpu.make_async_copy(dst_hbm.at[pl.ds(0, _CH)], dstv[sl],
                              isem[sl]).wait()

    def issue_gather(sl, cc):
        base = ebase + cc * _CH
        pltpu.async_copy(kv_hbm.at[srcv[sl]], kvb_[sl], gsem[sl])
        pltpu.async_copy(q_hbm.at[dstv[sl]], qb_[sl], gsem[sl])
        pltpu.async_copy(ep_hbm.at[pl.ds(base, _CH)], eb_[sl], gsem[sl])

    def wait_gather(sl):
        pltpu.make_async_copy(kv_hbm.at[srcv[sl]], kvb_[sl], gsem[sl]).wait()
        pltpu.make_async_copy(q_hbm.at[dstv[sl]], qb_[sl], gsem[sl]).wait()
        pltpu.make_async_copy(ep_hbm.at[pl.ds(0, _CH)], eb_[sl],
                              gsem[sl]).wait()

    def issue_scatter(sl):
        pltpu.async_copy(mb_[sl], nacc.at[dstsv[sl]], ssem[sl], add=True)
        pltpu.async_copy(db_[sl], dacc.at[dhiv[sl]], ssem[sl], add=True)

    def wait_scatter(sl):
        pltpu.make_async_copy(mb_[sl], nacc.at[dstsv[sl]], ssem[sl]).wait()
        pltpu.make_async_copy(db_[sl], dacc.at[dhiv[sl]], ssem[sl]).wait()

    def compute(sl, colv):
        qb, kvb, eb, mb, db = (qb_[sl], kvb_[sl], eb_[sl], mb_[sl], db_[sl])
        oldcolv = colvb[sl][...]
        colvb[sl][...] = colv
        for j in range(_CH):
            ps = []
            us = []
            for h in range(H):
                slc = pl.ds(h * C, C)
                e = eb[j, slc]
                ps.append(qb[j, slc] * (kvb[j, slc] + e))
                us.append(kvb[j, pl.ds(D + h * C, C)] + e)
            z1 = comb(ps[0], ps[1], 8)
            z2 = comb(ps[2], ps[3], 8)
            z3 = comb(ps[4], ps[5], 8)
            z4 = comb(ps[6], ps[7], 8)
            za = comb(z1, z2, 4)
            zb = comb(z3, z4, 4)
            zf = comb(za, zb, 2)
            av = zf + _shuf(zf, rp[1])
            ex = jnp.exp(av)
            db[j, pl.ds(oldcolv[j], C)] = zero16
            db[j, pl.ds(colv[j], C)] = ex
            for h in range(H):
                mb[j, pl.ds(h * C, C)] = us[h] * ex[_LANE_OF[h]]

    issue_idx(0, 0)
    issue_idx(1, 1)
    wait_idx(0)
    issue_gather(0, 0)
    wait_idx(1)
    issue_gather(1, 1)
    wait_gather(0)
    d16 = dstv[0][...]
    dstsv[0][...] = d16
    dhiv[0][...] = lax.shift_right_logical(d16, 3)
    colv0 = (d16 & 7) * C
    issue_idx(0, 2)
    compute(0, colv0)
    issue_scatter(0)

    def pair_body(tt, _):
        cc1 = 1 + 2 * tt
        @pl.when(cc1 + 1 <= _NCH - 1)
        def _():
            wait_idx(0)
            issue_gather(0, cc1 + 1)
        wait_scatter(0)
        wait_gather(1)
        d16a = dstv[1][...]
        dstsv[1][...] = d16a
        dhiv[1][...] = lax.shift_right_logical(d16a, 3)
        colva = (d16a & 7) * C

        @pl.when(cc1 + 2 <= _NCH - 1)
        def _():
            issue_idx(1, cc1 + 2)
        compute(1, colva)
        issue_scatter(1)

        cc2 = cc1 + 1

        @pl.when(cc2 + 1 <= _NCH - 1)
        def _():
            wait_idx(1)
            issue_gather(1, cc2 + 1)
        wait_scatter(1)
        wait_gather(0)
        d16b = dstv[0][...]
        dstsv[0][...] = d16b
        dhiv[0][...] = lax.shift_right_logical(d16b, 3)
        colvb2 = (d16b & 7) * C

        @pl.when(cc2 + 2 <= _NCH - 1)
        def _():
            issue_idx(0, cc2 + 2)
        compute(0, colvb2)
        issue_scatter(0)
        return 0

    lax.fori_loop(0, (_NCH - 1) // 2, pair_body, 0)
    wait_scatter(0)
    plsc.subcore_barrier()
    pltpu.sync_copy(nacc.at[pl.ds(sid * _RPT, _RPT)],
                    num_hbm.at[cid, pl.ds(sid * _RPT, _RPT)])
    pltpu.sync_copy(dacc.at[pl.ds(sid * _DRT, _DRT)],
                    den_hbm.at[cid, pl.ds(sid * _DRT, _DRT)])


def _edge_attention(q, kv, ep, src, dst):
    mesh = plsc.VectorSubcoreMesh(core_axis_name="c", subcore_axis_name="s")
    zeros_hbm = jnp.zeros((_RPT, D), jnp.float32)
    i16 = lambda: pltpu.VMEM((_CH,), jnp.int32)
    f16 = lambda: pltpu.VMEM((_CH, D), jnp.float32)
    f32w = lambda: pltpu.VMEM((_CH, 2 * D), jnp.float32)
    kern = functools.partial(
        pl.kernel,
        mesh=mesh,
        out_type=[
            jax.ShapeDtypeStruct((2, _NP, D), jnp.float32),
            jax.ShapeDtypeStruct((2, _DR, D), jnp.float32),
        ],
        scratch_types=(
            [i16() for _ in range(10)]
            + [f16(), f16(), f32w(), f32w()]
            + [f16() for _ in range(6)] + [
                pltpu.VMEM_SHARED((_NP, D), jnp.float32),
                pltpu.VMEM_SHARED((_DR, D), jnp.float32),
            ] + [pltpu.SemaphoreType.DMA] * 6
        ),
    )(_edge_kernel_body)
    return kern(q, kv, ep, src, dst, zeros_hbm)


def _post_body(num_ref, den_ref, x_ref, xr_ref, wa, wb, erep, h_o, stats_o):
    i = pl.program_id(0)
    o = num_ref[0] + num_ref[1]
    den = den_ref[0] + den_ref[1]
    den128 = jnp.zeros((_BN, D), jnp.float32)
    for hh in range(H):
        lh = _LANE_OF[hh]
        den128 = den128 + (1.0 / (den[:, lh:lh + 1] + 1e-16)) * erep[hh:hh + 1, :]
    out = o * den128
    xr = xr_ref[...]
    xb = x_ref[...]
    ba = jnp.sum(out * wa[...], axis=1, keepdims=True)
    bb = jnp.sum(xr * wb[...], axis=1, keepdims=True)
    b = jax.nn.sigmoid(ba + bb)
    hcur = xb + b * xr + (1.0 - b) * out
    h_o[...] = hcur

    @pl.when(i == 0)
    def _():
        stats_o[...] = jnp.zeros_like(stats_o)
    stats_o[0:1, :] += jnp.sum(hcur, axis=0, keepdims=True)
    stats_o[1:2, :] += jnp.sum(hcur * hcur, axis=0, keepdims=True)


def _post(num, den, x, xr, wa, wb, erep):
    return pl.pallas_call(
        _post_body,
        grid=(N // _BN,),
        in_specs=[
            pl.BlockSpec((2, _BN, D), lambda i: (0, i, 0)),
            pl.BlockSpec((2, _BN, C), lambda i: (0, i, 0)),
            pl.BlockSpec((_BN, D), lambda i: (i, 0)),
            pl.BlockSpec((_BN, D), lambda i: (i, 0)),
            pl.BlockSpec((1, D), lambda i: (0, 0)),
            pl.BlockSpec((1, D), lambda i: (0, 0)),
            pl.BlockSpec((H, D), lambda i: (0, 0)),
        ],
        out_specs=[
            pl.BlockSpec((_BN, D), lambda i: (i, 0)),
            pl.BlockSpec((8, D), lambda i: (0, 0)),
        ],
        out_shape=[
            jax.ShapeDtypeStruct((N, D), jnp.float32),
            jax.ShapeDtypeStruct((8, D), jnp.float32),
        ],
    )(num, den, x, xr, wa, wb, erep)


_SQRT_HALF = 0.7071067811865476


def _ffn_body(h_ref, stats_ref, gnw, gnb, gms, w1, b1, w2, b2, o_ref):
    s1 = stats_ref[0:1, :]
    s2 = stats_ref[1:2, :]
    m = s1 * (1.0 / N)
    q2 = s2 * (1.0 / N)
    g = gms[...]
    var = q2 - (2.0 * g - g * g) * m * m
    inv = lax.rsqrt(var + 1e-5)
    hb = h_ref[...]
    o = hb - g * m
    hn = gnw[...] * o * inv + gnb[...]
    t = jnp.dot(hn, w1[...], preferred_element_type=jnp.float32) + b1[...]
    gel = t * 0.5 * (1.0 + lax.erf(t * _SQRT_HALF))
    f = jnp.dot(gel, w2[...], preferred_element_type=jnp.float32) + b2[...]
    o_ref[...] = hn + f


def _ffn(h, stats, gnw, gnb, gms, w1, b1, w2, b2):
    return pl.pallas_call(
        _ffn_body,
        grid=(N // _BN,),
        in_specs=[
            pl.BlockSpec((_BN, D), lambda i: (i, 0)),
            pl.BlockSpec((8, D), lambda i: (0, 0)),
            pl.BlockSpec((1, D), lambda i: (0, 0)),
            pl.BlockSpec((1, D), lambda i: (0, 0)),
            pl.BlockSpec((1, D), lambda i: (0, 0)),
            pl.BlockSpec((D, FF), lambda i: (0, 0)),
            pl.BlockSpec((1, FF), lambda i: (0, 0)),
            pl.BlockSpec((FF, D), lambda i: (0, 0)),
            pl.BlockSpec((1, D), lambda i: (0, 0)),
        ],
        out_specs=pl.BlockSpec((_BN, D), lambda i: (i, 0)),
        out_shape=jax.ShapeDtypeStruct((N, D), jnp.float32),
    )(h, stats, gnw, gnb, gms, w1, b1, w2, b2)


def kernel(x, edge_index, edge_attr, params):
    p = params
    row = lambda a: a.reshape(1, -1)
    q, kv, xr = _proj(
        x, p['Wq'] * 0.25, row(p['bq'] * 0.25), p['Wk'], row(p['bk']),
        p['Wv'], row(p['bv']), p['Wskip'], row(p['bskip']))
    ep = _eproj(edge_attr.T, p['We'], row(p['be']))
    src = edge_index[0]
    dst = edge_index[1]
    num, den = _edge_attention(q, kv, ep, src, dst)
    den = den.reshape(2, _NP, C)

    wbeta = p['Wbeta'][:, 0]
    wa = row(wbeta[:D] + wbeta[2 * D:])
    wb = row(wbeta[D:2 * D] - wbeta[2 * D:])
    erep = jnp.repeat(jnp.eye(H, dtype=jnp.float32), C, axis=1)
    h, stats = _post(num, den, x, xr, wa, wb, erep)
    return _ffn(h, stats, row(p['gn_weight']), row(p['gn_bias']),
                row(p['gn_mean_scale']), p['W1'], row(p['b1']),
                p['W2'], row(p['b2']))


# speedup vs baseline: 1.2454x; 1.1440x over previous
import functools

import jax
import jax.numpy as jnp
import numpy as np
from jax import lax
from jax.experimental import pallas as pl
from jax.experimental.pallas import tpu as pltpu
from jax.experimental.pallas import tpu_sc as plsc

N = 10000
E = 320000
D = 128
H = 8
C = 16
ED = 10
FF = 4 * D

_BN = 1000


def _proj_body(x_ref, wq, bq, wk, bk, wv, bv, ws, bs, q_o, kv_o, xr_o):
    xb = x_ref[...]
    q_o[...] = jnp.dot(xb, wq[...], preferred_element_type=jnp.float32) + bq[...]
    kv_o[:, :D] = jnp.dot(xb, wk[...], preferred_element_type=jnp.float32) + bk[...]
    kv_o[:, D:] = jnp.dot(xb, wv[...], preferred_element_type=jnp.float32) + bv[...]
    xr_o[...] = jnp.dot(xb, ws[...], preferred_element_type=jnp.float32) + bs[...]


def _proj(x, wq, bq, wk, bk, wv, bv, ws, bs):
    blk = pl.BlockSpec((_BN, D), lambda i: (i, 0))
    wspec = pl.BlockSpec((D, D), lambda i: (0, 0))
    bspec = pl.BlockSpec((1, D), lambda i: (0, 0))
    return pl.pallas_call(
        _proj_body,
        grid=(N // _BN,),
        in_specs=[blk, wspec, bspec, wspec, bspec, wspec, bspec, wspec, bspec],
        out_specs=[blk, pl.BlockSpec((_BN, 2 * D), lambda i: (i, 0)), blk],
        out_shape=[jax.ShapeDtypeStruct((N, D), jnp.float32),
                   jax.ShapeDtypeStruct((N, 2 * D), jnp.float32),
                   jax.ShapeDtypeStruct((N, D), jnp.float32)],
    )(x, wq, bq, wk, bk, wv, bv, ws, bs)


_BE = 2560


def _eproj_body(eat_ref, we, be, o_ref):
    o_ref[...] = lax.dot_general(
        eat_ref[...], we[...], (((0,), (0,)), ((), ())),
        preferred_element_type=jnp.float32) + be[...]


def _eproj(eat, we, be):
    return pl.pallas_call(
        _eproj_body,
        grid=(E // _BE,),
        in_specs=[
            pl.BlockSpec((ED, _BE), lambda i: (0, i)),
            pl.BlockSpec((ED, D), lambda i: (0, 0)),
            pl.BlockSpec((1, D), lambda i: (0, 0)),
        ],
        out_specs=pl.BlockSpec((_BE, D), lambda i: (i, 0)),
        out_shape=jax.ShapeDtypeStruct((E, D), jnp.float32),
    )(eat, we, be)


_GDN = lax.GatherDimensionNumbers(
    offset_dims=(), collapsed_slice_dims=(0,), start_index_map=(0,))


def _shuf(v, idx):
    return lax.gather(v, idx[:, None], _GDN, slice_sizes=(1,),
                      mode=lax.GatherScatterMode.PROMISE_IN_BOUNDS)


_NTILE = 16
_NW = 2 * _NTILE
_EPW = E // _NW
_CH = 16
_NCH = _EPW // _CH
_NP = 10240
_RPT = _NP // _NTILE
_DR = _NP // 8
_DRT = _DR // _NTILE

_LANE_OF = (0, 8, 4, 12, 2, 10, 6, 14)


def _edge_kernel_body(q_hbm, kv_hbm, ep_hbm, src_hbm, dst_hbm, z_hbm,
                      num_hbm, den_hbm,
                      src0, src1, dst0, dst1, dsts0, dsts1, dhi0, dhi1,
                      cv0, cv1,
                      qb0, qb1, kvb0, kvb1, eb0, eb1,
                      mb0, mb1, db0, db1, nacc, dacc,
                      isem0, isem1, gsem0, gsem1, ssem0, ssem1):
    cid = lax.axis_index("c")
    sid = lax.axis_index("s")
    wid = cid * _NTILE + sid
    ebase = wid * _EPW

    srcv = (src0, src1)
    dstv = (dst0, dst1)
    dstsv = (dsts0, dsts1)
    dhiv = (dhi0, dhi1)
    colvb = (cv0, cv1)
    qb_ = (qb0, qb1)
    kvb_ = (kvb0, kvb1)
    eb_ = (eb0, eb1)
    mb_ = (mb0, mb1)
    db_ = (db0, db1)
    isem = (isem0, isem1)
    gsem = (gsem0, gsem1)
    ssem = (ssem0, ssem1)

    pltpu.sync_copy(z_hbm.at[pl.ds(0, _RPT)],
                    nacc.at[pl.ds(sid * _RPT, _RPT)])
    pltpu.sync_copy(z_hbm.at[pl.ds(0, _DRT)], dacc.at[pl.ds(sid * _DRT, _DRT)])
    zero16 = jnp.zeros((C,), jnp.float32)
    zi16 = jnp.zeros((C,), jnp.int32)
    for j in range(_CH):
        for bq in range(H):
            db0[j, pl.ds(bq * C, C)] = zero16
            db1[j, pl.ds(bq * C, C)] = zero16
    cv0[...] = zi16
    cv1[...] = zi16
    plsc.subcore_barrier()

    lane = lax.iota(jnp.int32, C)
    rp = {sh: (lane + sh) % C for sh in (1, 2, 4, 8)}
    rm = {sh: (lane - sh) % C for sh in (2, 4)}
    msk = {sh: (lane & sh) == 0 for sh in (2, 4, 8)}

    def comb(A, B, sh):
        rB = rp[sh] if sh == 8 else rm[sh]
        return jnp.where(msk[sh], A + _shuf(A, rp[sh]), B + _shuf(B, rB))

    def issue_idx(sl, cc):
        base = ebase + cc * _CH
        pltpu.async_copy(src_hbm.at[pl.ds(base, _CH)], srcv[sl], isem[sl])
        pltpu.async_copy(dst_hbm.at[pl.ds(base, _CH)], dstv[sl], isem[sl])

    def wait_idx(sl):
        pltpu.make_async_copy(src_hbm.at[pl.ds(0, _CH)], srcv[sl],
                              isem[sl]).wait()
        pltpu.make_async_copy(dst_hbm.at[pl.ds(0, _CH)], dstv[sl],
                              isem[sl]).wait()

    def issue_gather(sl, cc):
        base = ebase + cc * _CH
        pltpu.async_copy(kv_hbm.at[srcv[sl]], kvb_[sl], gsem[sl])
        pltpu.async_copy(q_hbm.at[dstv[sl]], qb_[sl], gsem[sl])
        pltpu.async_copy(ep_hbm.at[pl.ds(base, _CH)], eb_[sl], gsem[sl])

    def wait_gather(sl):
        pltpu.make_async_copy(kv_hbm.at[srcv[sl]], kvb_[sl], gsem[sl]).wait()
        pltpu.make_async_copy(q_hbm.at[dstv[sl]], qb_[sl], gsem[sl]).wait()
        pltpu.make_async_copy(ep_hbm.at[pl.ds(0, _CH)], eb_[sl],
                              gsem[sl]).wait()

    def issue_scatter(sl):
        pltpu.async_copy(mb_[sl], nacc.at[dstsv[sl]], ssem[sl], add=True)
        pltpu.async_copy(db_[sl], dacc.at[dhiv[sl]], ssem[sl], add=True)

    def wait_scatter(sl):
        pltpu.make_async_copy(mb_[sl], nacc.at[dstsv[sl]], ssem[sl]).wait()
        pltpu.make_async_copy(db_[sl], dacc.at[dhiv[sl]], ssem[sl]).wait()

    def compute(sl, colv):
        qb, kvb, eb, mb, db = (qb_[sl], kvb_[sl], eb_[sl], mb_[sl], db_[sl])
        oldcolv = colvb[sl][...]
        colvb[sl][...] = colv
        st = {}
        exs = {}
        for j in range(_CH + 2):
            if j < _CH:
                ps = []
                us = []
                for h in range(H):
                    slc = pl.ds(h * C, C)
                    e = eb[j, slc]
                    ps.append(qb[j, slc] * (kvb[j, slc] + e))
                    us.append(kvb[j, pl.ds(D + h * C, C)] + e)
                st[j] = (ps, us)
            j1 = j - 1
            if 0 <= j1 < _CH:
                ps, us = st[j1]
                z1 = comb(ps[0], ps[1], 8)
                z2 = comb(ps[2], ps[3], 8)
                z3 = comb(ps[4], ps[5], 8)
                z4 = comb(ps[6], ps[7], 8)
                za = comb(z1, z2, 4)
                zb = comb(z3, z4, 4)
                zf = comb(za, zb, 2)
                av = zf + _shuf(zf, rp[1])
                exs[j1] = jnp.exp(av)
            j2 = j - 2
            if 0 <= j2 < _CH:
                ex = exs.pop(j2)
                us = st.pop(j2)[1]
                db[j2, pl.ds(oldcolv[j2], C)] = zero16
                db[j2, pl.ds(colv[j2], C)] = ex
                for h in range(H):
                    mb[j2, pl.ds(h * C, C)] = us[h] * ex[_LANE_OF[h]]

    issue_idx(0, 0)
    issue_idx(1, 1)
    wait_idx(0)
    issue_gather(0, 0)
    wait_idx(1)
    issue_gather(1, 1)
    wait_gather(0)
    d16 = dstv[0][...]
    dstsv[0][...] = d16
    dhiv[0][...] = lax.shift_right_logical(d16, 3)
    colv0 = (d16 & 7) * C
    issue_idx(0, 2)
    compute(0, colv0)
    issue_scatter(0)

    def pair_body(tt, _):
        cc1 = 1 + 2 * tt
        @pl.when(cc1 + 1 <= _NCH - 1)
        def _():
            wait_idx(0)
            issue_gather(0, cc1 + 1)
        wait_scatter(0)
        wait_gather(1)
        d16a = dstv[1][...]
        dstsv[1][...] = d16a
        dhiv[1][...] = lax.shift_right_logical(d16a, 3)
        colva = (d16a & 7) * C

        @pl.when(cc1 + 2 <= _NCH - 1)
        def _():
            issue_idx(1, cc1 + 2)
        compute(1, colva)
        issue_scatter(1)

        cc2 = cc1 + 1

        @pl.when(cc2 + 1 <= _NCH - 1)
        def _():
            wait_idx(1)
            issue_gather(1, cc2 + 1)
        wait_scatter(1)
        wait_gather(0)
        d16b = dstv[0][...]
        dstsv[0][...] = d16b
        dhiv[0][...] = lax.shift_right_logical(d16b, 3)
        colvb2 = (d16b & 7) * C

        @pl.when(cc2 + 2 <= _NCH - 1)
        def _():
            issue_idx(0, cc2 + 2)
        compute(0, colvb2)
        issue_scatter(0)
        return 0

    lax.fori_loop(0, (_NCH - 1) // 2, pair_body, 0)
    wait_scatter(0)
    plsc.subcore_barrier()
    pltpu.sync_copy(nacc.at[pl.ds(sid * _RPT, _RPT)],
                    num_hbm.at[cid, pl.ds(sid * _RPT, _RPT)])
    pltpu.sync_copy(dacc.at[pl.ds(sid * _DRT, _DRT)],
                    den_hbm.at[cid, pl.ds(sid * _DRT, _DRT)])


def _edge_attention(q, kv, ep, src, dst):
    mesh = plsc.VectorSubcoreMesh(core_axis_name="c", subcore_axis_name="s")
    zeros_hbm = jnp.zeros((_RPT, D), jnp.float32)
    i16 = lambda: pltpu.VMEM((_CH,), jnp.int32)
    f16 = lambda: pltpu.VMEM((_CH, D), jnp.float32)
    f32w = lambda: pltpu.VMEM((_CH, 2 * D), jnp.float32)
    kern = functools.partial(
        pl.kernel,
        mesh=mesh,
        out_type=[
            jax.ShapeDtypeStruct((2, _NP, D), jnp.float32),
            jax.ShapeDtypeStruct((2, _DR, D), jnp.float32),
        ],
        scratch_types=(
            [i16() for _ in range(10)]
            + [f16(), f16(), f32w(), f32w()]
            + [f16() for _ in range(6)] + [
                pltpu.VMEM_SHARED((_NP, D), jnp.float32),
                pltpu.VMEM_SHARED((_DR, D), jnp.float32),
            ] + [pltpu.SemaphoreType.DMA] * 6
        ),
    )(_edge_kernel_body)
    return kern(q, kv, ep, src, dst, zeros_hbm)


def _post_body(num_ref, den_ref, x_ref, xr_ref, wa, wb, erep, h_o, stats_o):
    i = pl.program_id(0)
    o = num_ref[0] + num_ref[1]
    den = den_ref[0] + den_ref[1]
    den128 = jnp.zeros((_BN, D), jnp.float32)
    for hh in range(H):
        lh = _LANE_OF[hh]
        den128 = den128 + (1.0 / (den[:, lh:lh + 1] + 1e-16)) * erep[hh:hh + 1, :]
    out = o * den128
    xr = xr_ref[...]
    xb = x_ref[...]
    ba = jnp.sum(out * wa[...], axis=1, keepdims=True)
    bb = jnp.sum(xr * wb[...], axis=1, keepdims=True)
    b = jax.nn.sigmoid(ba + bb)
    hcur = xb + b * xr + (1.0 - b) * out
    h_o[...] = hcur

    @pl.when(i == 0)
    def _():
        stats_o[...] = jnp.zeros_like(stats_o)
    stats_o[0:1, :] += jnp.sum(hcur, axis=0, keepdims=True)
    stats_o[1:2, :] += jnp.sum(hcur * hcur, axis=0, keepdims=True)


def _post(num, den, x, xr, wa, wb, erep):
    return pl.pallas_call(
        _post_body,
        grid=(N // _BN,),
        in_specs=[
            pl.BlockSpec((2, _BN, D), lambda i: (0, i, 0)),
            pl.BlockSpec((2, _BN, C), lambda i: (0, i, 0)),
            pl.BlockSpec((_BN, D), lambda i: (i, 0)),
            pl.BlockSpec((_BN, D), lambda i: (i, 0)),
            pl.BlockSpec((1, D), lambda i: (0, 0)),
            pl.BlockSpec((1, D), lambda i: (0, 0)),
            pl.BlockSpec((H, D), lambda i: (0, 0)),
        ],
        out_specs=[
            pl.BlockSpec((_BN, D), lambda i: (i, 0)),
            pl.BlockSpec((8, D), lambda i: (0, 0)),
        ],
        out_shape=[
            jax.ShapeDtypeStruct((N, D), jnp.float32),
            jax.ShapeDtypeStruct((8, D), jnp.float32),
        ],
    )(num, den, x, xr, wa, wb, erep)


_SQRT_HALF = 0.7071067811865476


def _ffn_body(h_ref, stats_ref, gnw, gnb, gms, w1, b1, w2, b2, o_ref):
    s1 = stats_ref[0:1, :]
    s2 = stats_ref[1:2, :]
    m = s1 * (1.0 / N)
    q2 = s2 * (1.0 / N)
    g = gms[...]
    var = q2 - (2.0 * g - g * g) * m * m
    inv = lax.rsqrt(var + 1e-5)
    hb = h_ref[...]
    o = hb - g * m
    hn = gnw[...] * o * inv + gnb[...]
    t = jnp.dot(hn, w1[...], preferred_element_type=jnp.float32) + b1[...]
    gel = t * 0.5 * (1.0 + lax.erf(t * _SQRT_HALF))
    f = jnp.dot(gel, w2[...], preferred_element_type=jnp.float32) + b2[...]
    o_ref[...] = hn + f


def _ffn(h, stats, gnw, gnb, gms, w1, b1, w2, b2):
    return pl.pallas_call(
        _ffn_body,
        grid=(N // _BN,),
        in_specs=[
            pl.BlockSpec((_BN, D), lambda i: (i, 0)),
            pl.BlockSpec((8, D), lambda i: (0, 0)),
            pl.BlockSpec((1, D), lambda i: (0, 0)),
            pl.BlockSpec((1, D), lambda i: (0, 0)),
            pl.BlockSpec((1, D), lambda i: (0, 0)),
            pl.BlockSpec((D, FF), lambda i: (0, 0)),
            pl.BlockSpec((1, FF), lambda i: (0, 0)),
            pl.BlockSpec((FF, D), lambda i: (0, 0)),
            pl.BlockSpec((1, D), lambda i: (0, 0)),
        ],
        out_specs=pl.BlockSpec((_BN, D), lambda i: (i, 0)),
        out_shape=jax.ShapeDtypeStruct((N, D), jnp.float32),
    )(h, stats, gnw, gnb, gms, w1, b1, w2, b2)


def kernel(x, edge_index, edge_attr, params):
    p = params
    row = lambda a: a.reshape(1, -1)
    q, kv, xr = _proj(
        x, p['Wq'] * 0.25, row(p['bq'] * 0.25), p['Wk'], row(p['bk']),
        p['Wv'], row(p['bv']), p['Wskip'], row(p['bskip']))
    ep = _eproj(edge_attr.T, p['We'], row(p['be']))
    src = edge_index[0]
    dst = edge_index[1]
    num, den = _edge_attention(q, kv, ep, src, dst)
    den = den.reshape(2, _NP, C)

    wbeta = p['Wbeta'][:, 0]
    wa = row(wbeta[:D] + wbeta[2 * D:])
    wb = row(wbeta[D:2 * D] - wbeta[2 * D:])
    erep = jnp.repeat(jnp.eye(H, dtype=jnp.float32), C, axis=1)
    h, stats = _post(num, den, x, xr, wa, wb, erep)
    return _ffn(h, stats, row(p['gn_weight']), row(p['gn_bias']),
                row(p['gn_mean_scale']), p['W1'], row(p['b1']),
                p['W2'], row(p['b2']))
